# baseline (speedup 1.0000x reference)
import functools

import jax
import jax.numpy as jnp
from jax import lax
from jax.experimental import pallas as pl
from jax.experimental.pallas import tpu as pltpu
from jax.experimental.pallas import tpu_sc as plsc

K = 1024
L = 16
CPW = 8
NPR = CPW // 2
P = 1792
NINF2 = -8323200

_INFO = plsc.get_sparse_core_info()
_NC, _NS = _INFO.num_cores, _INFO.num_subcores
NW = _NC * _NS


@functools.lru_cache(maxsize=None)
def _build(B, C, HW):
    NU = (B * C) // CPW
    UPW = NU // NW
    NCH = HW // P
    assert (B * C) % CPW == 0 and NU % NW == 0
    assert HW % P == 0 and NCH % 2 == 0 and P % L == 0

    mesh = plsc.VectorSubcoreMesh(core_axis_name="c", subcore_axis_name="s")
    scratch = (
        [pltpu.VMEM((P,), jnp.int32) for _ in range(2)]
        + [pltpu.VMEM((NPR * P,), jnp.int32) for _ in range(2)]
        + [pltpu.VMEM((L * K,), jnp.int32) for _ in range(NPR)]
        + [pltpu.VMEM((K,), jnp.float32) for _ in range(2)]
        + [pltpu.SemaphoreType.DMA, pltpu.SemaphoreType.DMA]
    )

    @functools.partial(
        pl.kernel,
        out_type=jax.ShapeDtypeStruct((B * C, K), jnp.float32),
        mesh=mesh,
        scratch_types=scratch,
        compiler_params=pltpu.CompilerParams(needs_layout_passes=False),
    )
    def k(pk_hbm, spx_hbm, out_hbm, *scr):
        labs = scr[0:2]
        dats = scr[2:4]
        accs = scr[4:4 + NPR]
        outs = scr[4 + NPR:6 + NPR]
        sems = scr[6 + NPR:8 + NPR]

        wid = lax.axis_index("s") * _NC + lax.axis_index("c")
        u0 = wid * UPW
        bankoff = lax.iota(jnp.int32, L) * K
        ninf2 = jnp.full((L,), NINF2, jnp.int32)

        upb = NU // B

        def issue(u, chunk, pbuf):
            off = chunk * P
            b = lax.div(u, upb)
            r0 = lax.rem(u, upb) * NPR + b * (upb * NPR)
            pltpu.async_copy(spx_hbm.at[b, pl.ds(off, P)], labs[pbuf], sems[pbuf])
            for t in range(NPR):
                pltpu.async_copy(pk_hbm.at[r0 + t, pl.ds(off, P)],
                                 dats[pbuf].at[pl.ds(t * P, P)], sems[pbuf])

        def drain(pbuf):
            pltpu.make_async_copy(
                spx_hbm.at[0, pl.ds(0, P)], labs[pbuf], sems[pbuf]).wait()
            for t in range(NPR):
                pltpu.make_async_copy(
                    pk_hbm.at[0, pl.ds(0, P)],
                    dats[pbuf].at[pl.ds(t * P, P)], sems[pbuf]).wait()

        def compute(pbuf):
            lab_ref = labs[pbuf]
            dat_ref = dats[pbuf]

            def vbody(v, carry):
                base = v * L
                idx = lab_ref[pl.ds(base, L)] + bankoff
                dv = [dat_ref[pl.ds(t * P + base, L)] for t in range(NPR)]
                gv = [plsc.load_gather(accs[t], [idx]) for t in range(NPR)]
                mv = [jnp.maximum(plsc.bitcast(g, jnp.bfloat16),
                                  plsc.bitcast(d, jnp.bfloat16))
                      for g, d in zip(gv, dv)]
                for t in range(NPR):
                    plsc.store_scatter(accs[t], [idx],
                                       plsc.bitcast(mv[t], jnp.int32))
                return carry

            lax.fori_loop(0, P // L, vbody, 0, unroll=2)

        def init_accs():
            def ibody(i, carry):
                base = i * L
                for t in range(NPR):
                    accs[t][pl.ds(base, L)] = ninf2
                return carry

            lax.fori_loop(0, (L * K) // L, ibody, 0)

        def merge_and_reset(u):
            b = lax.div(u, upb)
            cg = lax.rem(u, upb)
            row0 = b * C + cg * CPW
            for t in range(NPR):
                def mbody(kv, carry):
                    base = kv * L
                    vals = [plsc.bitcast(accs[t][pl.ds(l * K + base, L)],
                                         jnp.bfloat16) for l in range(L)]
                    while len(vals) > 1:
                        vals = [jnp.maximum(vals[i], vals[i + 1])
                                for i in range(0, len(vals), 2)]
                    lo, hi = plsc.unpack(vals[0],
                                         format=plsc.PackFormat.INTERLEAVED)
                    outs[0][pl.ds(base, L)] = lo
                    outs[1][pl.ds(base, L)] = hi
                    for l in range(L):
                        accs[t][pl.ds(l * K + base, L)] = ninf2
                    return carry

                lax.fori_loop(0, K // L, mbody, 0)
                pltpu.sync_copy(outs[0], out_hbm.at[row0 + 2 * t])
                pltpu.sync_copy(outs[1], out_hbm.at[row0 + 2 * t + 1])

        init_accs()
        issue(u0, 0, 0)
        for du in range(UPW):
            u = u0 + du

            def gbody(g, carry):
                issue(u, 2 * g + 1, 1)
                drain(0)
                compute(0)
                issue(u, 2 * g + 2, 0)
                drain(1)
                compute(1)
                return carry

            lax.fori_loop(0, NCH // 2 - 1, gbody, 0)
            issue(u, NCH - 1, 1)
            drain(0)
            compute(0)
            if du < UPW - 1:
                issue(u + 1, 0, 0)
            drain(1)
            compute(1)
            merge_and_reset(u)

    return k


def kernel(img, spx):
    B, C, H, W = img.shape
    HW = H * W
    spx2 = spx.reshape(B, HW).astype(jnp.int32)
    CB = 64 if (C % 64 == 0 and (B * 64) % (CPW * NW) == 0) else C
    build = _build(B, CB, HW)
    outs = []
    for c0 in range(0, C, CB):
        blk = img[:, c0:c0 + CB]
        pairs = blk.astype(jnp.bfloat16).reshape(B, CB // 2, 2, HW)
        pairs = pairs.transpose(0, 1, 3, 2)
        packed = lax.bitcast_convert_type(
            pairs, jnp.int32).reshape(B * CB // 2, HW)
        outs.append(build(packed, spx2).reshape(B, CB, K))
    return jnp.concatenate(outs, axis=1) if len(outs) > 1 else outs[0]


# speedup vs baseline: 2.1741x; 2.1741x over previous
import functools

import jax
import jax.numpy as jnp
from jax import lax
from jax.experimental import pallas as pl
from jax.experimental.pallas import tpu as pltpu
from jax.experimental.pallas import tpu_sc as plsc

K = 1024
L = 16
CPW = 8
NPR = CPW // 2
P = 1792
NINF2 = -8323200

_INFO = plsc.get_sparse_core_info()
_NC, _NS = _INFO.num_cores, _INFO.num_subcores
NW = _NC * _NS


@functools.lru_cache(maxsize=None)
def _build(B, C, HW):
    NU = (B * C) // CPW
    UPW = NU // NW
    NCH = HW // P
    assert (B * C) % CPW == 0 and NU % NW == 0
    assert HW % P == 0 and NCH % 2 == 0 and P % L == 0

    mesh = plsc.VectorSubcoreMesh(core_axis_name="c", subcore_axis_name="s")
    scratch = (
        [pltpu.VMEM((P,), jnp.int32) for _ in range(2)]
        + [pltpu.VMEM((NPR * P,), jnp.int32) for _ in range(2)]
        + [pltpu.VMEM((L * K,), jnp.int32) for _ in range(NPR)]
        + [pltpu.VMEM((K,), jnp.float32) for _ in range(2)]
        + [pltpu.SemaphoreType.DMA, pltpu.SemaphoreType.DMA]
    )

    @functools.partial(
        pl.kernel,
        out_type=jax.ShapeDtypeStruct((B * C, K), jnp.float32),
        mesh=mesh,
        scratch_types=scratch,
        compiler_params=pltpu.CompilerParams(needs_layout_passes=False),
    )
    def k(pk_hbm, spx_hbm, out_hbm, *scr):
        labs = scr[0:2]
        dats = scr[2:4]
        accs = scr[4:4 + NPR]
        outs = scr[4 + NPR:6 + NPR]
        sems = scr[6 + NPR:8 + NPR]

        wid = lax.axis_index("s") * _NC + lax.axis_index("c")
        u0 = wid * UPW
        bankoff = lax.iota(jnp.int32, L) * K
        ninf2 = jnp.full((L,), NINF2, jnp.int32)

        upb = NU // B

        def issue(u, chunk, pbuf):
            off = chunk * P
            b = lax.div(u, upb)
            r0 = lax.rem(u, upb) * NPR + b * (upb * NPR)
            pltpu.async_copy(spx_hbm.at[b, pl.ds(off, P)], labs[pbuf], sems[pbuf])
            for t in range(NPR):
                pltpu.async_copy(pk_hbm.at[r0 + t, pl.ds(off, P)],
                                 dats[pbuf].at[pl.ds(t * P, P)], sems[pbuf])

        def drain(pbuf):
            pltpu.make_async_copy(
                spx_hbm.at[0, pl.ds(0, P)], labs[pbuf], sems[pbuf]).wait()
            for t in range(NPR):
                pltpu.make_async_copy(
                    pk_hbm.at[0, pl.ds(0, P)],
                    dats[pbuf].at[pl.ds(t * P, P)], sems[pbuf]).wait()

        def compute(pbuf):
            lab_ref = labs[pbuf]
            dat_ref = dats[pbuf]

            def vbody(v, carry):
                base = v * L
                idx = lab_ref[pl.ds(base, L)] + bankoff
                dv = [dat_ref[pl.ds(t * P + base, L)] for t in range(NPR)]
                gv = [plsc.load_gather(accs[t], [idx]) for t in range(NPR)]
                mv = [jnp.maximum(plsc.bitcast(g, jnp.bfloat16),
                                  plsc.bitcast(d, jnp.bfloat16))
                      for g, d in zip(gv, dv)]
                for t in range(NPR):
                    plsc.store_scatter(accs[t], [idx],
                                       plsc.bitcast(mv[t], jnp.int32))
                return carry

            lax.fori_loop(0, P // L, vbody, 0, unroll=2)

        def init_accs():
            def ibody(i, carry):
                base = i * L
                for t in range(NPR):
                    accs[t][pl.ds(base, L)] = ninf2
                return carry

            lax.fori_loop(0, (L * K) // L, ibody, 0)

        def merge_and_reset(u):
            b = lax.div(u, upb)
            cg = lax.rem(u, upb)
            row0 = b * C + cg * CPW
            for t in range(NPR):
                def mbody(kv, carry):
                    base = kv * L
                    vals = [plsc.bitcast(accs[t][pl.ds(l * K + base, L)],
                                         jnp.bfloat16) for l in range(L)]
                    while len(vals) > 1:
                        vals = [jnp.maximum(vals[i], vals[i + 1])
                                for i in range(0, len(vals), 2)]
                    lo, hi = plsc.unpack(vals[0],
                                         format=plsc.PackFormat.INTERLEAVED)
                    outs[0][pl.ds(base, L)] = lo
                    outs[1][pl.ds(base, L)] = hi
                    for l in range(L):
                        accs[t][pl.ds(l * K + base, L)] = ninf2
                    return carry

                lax.fori_loop(0, K // L, mbody, 0)
                pltpu.sync_copy(outs[0], out_hbm.at[row0 + 2 * t])
                pltpu.sync_copy(outs[1], out_hbm.at[row0 + 2 * t + 1])

        init_accs()
        issue(u0, 0, 0)
        for du in range(UPW):
            u = u0 + du

            def gbody(g, carry):
                issue(u, 2 * g + 1, 1)
                drain(0)
                compute(0)
                issue(u, 2 * g + 2, 0)
                drain(1)
                compute(1)
                return carry

            lax.fori_loop(0, NCH // 2 - 1, gbody, 0)
            issue(u, NCH - 1, 1)
            drain(0)
            compute(0)
            if du < UPW - 1:
                issue(u + 1, 0, 0)
            drain(1)
            compute(1)
            merge_and_reset(u)

    return k


def kernel(img, spx):
    B, C, H, W = img.shape
    HW = H * W
    spx2 = spx.reshape(B, HW).astype(jnp.int32)
    CB = 64 if (C % 64 == 0 and (B * 64) % (CPW * NW) == 0) else C
    build = _build(B, CB, HW)
    outs = []
    for c0 in range(0, C, CB):
        blk = img[:, c0:c0 + CB]
        lo = lax.bitcast_convert_type(
            blk[:, 0::2].astype(jnp.bfloat16), jnp.uint16).astype(jnp.uint32)
        hi = lax.bitcast_convert_type(
            blk[:, 1::2].astype(jnp.bfloat16), jnp.uint16).astype(jnp.uint32)
        packed = (lo | (hi << 16)).astype(jnp.int32).reshape(B * CB // 2, HW)
        outs.append(build(packed, spx2).reshape(B, CB, K))
    return jnp.concatenate(outs, axis=1) if len(outs) > 1 else outs[0]
